# baseline (speedup 1.0000x reference)
import jax
import jax.numpy as jnp
from jax.experimental import pallas as pl

_EPS = 1e-9


def _diversity_sum_kernel(x_ref, y_ref, out_ref):
    b = pl.program_id(0)
    x = x_ref[0]
    y = y_ref[0]
    dx = x[:, None, :] - x[None, :, :]
    dy = y[:, None, :] - y[None, :, :]
    d = jnp.sqrt(dx * dx + dy * dy + _EPS)
    s = jnp.sum(d)

    @pl.when(b == 0)
    def _():
        out_ref[:, :] = jnp.zeros_like(out_ref)

    out_ref[:, :] = out_ref[:, :] + s


def kernel(predicted_trajectory):
    traj = predicted_trajectory[..., :2]
    B, C, T = traj.shape[:3]
    x = traj[..., 0]
    y = traj[..., 1]
    total = pl.pallas_call(
        _diversity_sum_kernel,
        grid=(B,),
        in_specs=[
            pl.BlockSpec((1, C, T), lambda b: (b, 0, 0)),
            pl.BlockSpec((1, C, T), lambda b: (b, 0, 0)),
        ],
        out_specs=pl.BlockSpec((1, 1), lambda b: (0, 0)),
        out_shape=jax.ShapeDtypeStruct((1, 1), jnp.float32),
    )(x, y)[0, 0]
    offdiag = total / T - B * C * jnp.sqrt(jnp.float32(_EPS))
    return -(offdiag / (B * C * (C - 1)))


# speedup vs baseline: 1.8822x; 1.8822x over previous
import jax
import jax.numpy as jnp
from jax.experimental import pallas as pl

_EPS = 1e-9


def _diversity_sum_kernel(x_ref, y_ref, out_ref):
    b = pl.program_id(0)
    x = x_ref[0]
    y = y_ref[0]
    C, T = x.shape
    R = 8
    nc = C // R
    acc1 = jnp.zeros((R, R, T), jnp.float32)
    acc2 = jnp.zeros((R, R, T), jnp.float32)
    for ci in range(nc):
        xi = x[ci * R:(ci + 1) * R][:, None, :]
        yi = y[ci * R:(ci + 1) * R][:, None, :]
        for cj in range(ci, nc):
            xj = x[cj * R:(cj + 1) * R][None, :, :]
            yj = y[cj * R:(cj + 1) * R][None, :, :]
            dx = xi - xj
            dy = yi - yj
            d = jnp.sqrt(dx * dx + dy * dy + _EPS)
            if ci == cj:
                acc1 = acc1 + d
            else:
                acc2 = acc2 + d
    s = 2.0 * jnp.sum(acc2) + jnp.sum(acc1)

    @pl.when(b == 0)
    def _():
        out_ref[:, :] = jnp.zeros_like(out_ref)

    out_ref[:, :] = out_ref[:, :] + s


def kernel(predicted_trajectory):
    traj = predicted_trajectory[..., :2]
    B, C, T = traj.shape[:3]
    x = traj[..., 0]
    y = traj[..., 1]
    total = pl.pallas_call(
        _diversity_sum_kernel,
        grid=(B,),
        in_specs=[
            pl.BlockSpec((1, C, T), lambda b: (b, 0, 0)),
            pl.BlockSpec((1, C, T), lambda b: (b, 0, 0)),
        ],
        out_specs=pl.BlockSpec((1, 1), lambda b: (0, 0)),
        out_shape=jax.ShapeDtypeStruct((1, 1), jnp.float32),
    )(x, y)[0, 0]
    offdiag = total / T - B * C * jnp.sqrt(jnp.float32(_EPS))
    return -(offdiag / (B * C * (C - 1)))
